# Initial kernel scaffold; baseline (speedup 1.0000x reference)
import jax
import jax.numpy as jnp
from jax.experimental import pallas as pl


def kernel(input_tensor, genre_table, key_table, W1, b1, W2, b2):
    raise NotImplementedError("write your pallas kernel here")


# speedup vs baseline: 2.1817x; 2.1817x over previous
import functools

import jax
import jax.numpy as jnp
from jax import lax
from jax.experimental import pallas as pl
from jax.experimental.pallas import tpu as pltpu
from jax.experimental.pallas import tpu_sc as plsc

N = 16384
EMB = 128
H2 = 512
OUT = 256
NW = 32
RPW = N // NW
IDX_W = 128
CHUNKS = RPW // IDX_W

_sc_mesh = plsc.VectorSubcoreMesh(core_axis_name="c", subcore_axis_name="s")


@functools.partial(
    pl.kernel,
    mesh=_sc_mesh,
    out_type=(
        jax.ShapeDtypeStruct((N, EMB), jnp.float32),
        jax.ShapeDtypeStruct((N, EMB), jnp.float32),
    ),
    scratch_types=[
        pltpu.VMEM((CHUNKS, IDX_W), jnp.int32),
        pltpu.VMEM((RPW, EMB), jnp.float32),
        pltpu.SemaphoreType.DMA,
    ],
)
def _sc_gather(gtab, ktab, gidx, kidx, gout, kout, idx_v, rows_v, sem):
    wid = lax.axis_index("s") * 2 + lax.axis_index("c")
    row0 = wid * RPW
    blk0 = wid * CHUNKS

    def one_table(tab, out_hbm, idx_hbm):
        pltpu.sync_copy(idx_hbm.at[pl.ds(blk0, CHUNKS)], idx_v)
        copies = []
        for j in range(CHUNKS):
            copies.append(
                pltpu.async_copy(
                    tab.at[idx_v.at[j]], rows_v.at[pl.ds(j * IDX_W, IDX_W)], sem
                )
            )
        for c in copies:
            c.wait()
        pltpu.sync_copy(rows_v, out_hbm.at[pl.ds(row0, RPW)])

    one_table(gtab, gout, gidx)
    one_table(ktab, kout, kidx)


def _mlp_body(g_ref, k_ref, w1a_ref, w1b_ref, w2_ref, b1_ref, b2_ref, o_ref):
    h = jnp.dot(g_ref[...], w1a_ref[...], preferred_element_type=jnp.float32)
    h = h + jnp.dot(k_ref[...], w1b_ref[...], preferred_element_type=jnp.float32)
    h = jnp.maximum(h + b1_ref[...], 0.0)
    o_ref[...] = (
        jnp.dot(h, w2_ref[...], preferred_element_type=jnp.float32) + b2_ref[...]
    )


BLK = 2048


def _mlp(gbuf, kbuf, w1a_t, w1b_t, w2_t, b1, b2):
    return pl.pallas_call(
        _mlp_body,
        grid=(N // BLK,),
        in_specs=[
            pl.BlockSpec((BLK, EMB), lambda i: (i, 0)),
            pl.BlockSpec((BLK, EMB), lambda i: (i, 0)),
            pl.BlockSpec((EMB, H2), lambda i: (0, 0)),
            pl.BlockSpec((EMB, H2), lambda i: (0, 0)),
            pl.BlockSpec((H2, OUT), lambda i: (0, 0)),
            pl.BlockSpec((1, H2), lambda i: (0, 0)),
            pl.BlockSpec((1, OUT), lambda i: (0, 0)),
        ],
        out_specs=pl.BlockSpec((BLK, OUT), lambda i: (i, 0)),
        out_shape=jax.ShapeDtypeStruct((N, OUT), jnp.float32),
    )(gbuf, kbuf, w1a_t, w1b_t, w2_t, b1, b2)


def kernel(input_tensor, genre_table, key_table, W1, b1, W2, b2):
    g_idx = input_tensor[:, 0].reshape(N // IDX_W, IDX_W)
    k_idx = input_tensor[:, 1].reshape(N // IDX_W, IDX_W)
    gbuf, kbuf = _sc_gather(genre_table, key_table, g_idx, k_idx)
    w1a_t = W1[:, :EMB].T
    w1b_t = W1[:, EMB:].T
    w2_t = W2.T
    return _mlp(
        gbuf, kbuf, w1a_t, w1b_t, w2_t, b1.reshape(1, H2), b2.reshape(1, OUT)
    )


# speedup vs baseline: 2.1924x; 1.0049x over previous
import functools

import jax
import jax.numpy as jnp
from jax import lax
from jax.experimental import pallas as pl
from jax.experimental.pallas import tpu as pltpu
from jax.experimental.pallas import tpu_sc as plsc

N = 16384
EMB = 128
H2 = 512
OUT = 256
NW = 32
RPW = N // NW
IDX_W = 128
CHUNKS = RPW // IDX_W

_sc_mesh = plsc.VectorSubcoreMesh(core_axis_name="c", subcore_axis_name="s")


@functools.partial(
    pl.kernel,
    mesh=_sc_mesh,
    out_type=(
        jax.ShapeDtypeStruct((N, EMB), jnp.float32),
        jax.ShapeDtypeStruct((N, EMB), jnp.float32),
    ),
    scratch_types=[
        pltpu.VMEM((CHUNKS, IDX_W), jnp.int32),
        pltpu.VMEM((RPW, EMB), jnp.float32),
        pltpu.SemaphoreType.DMA,
    ],
)
def _sc_gather(gtab, ktab, gidx, kidx, gout, kout, idx_v, rows_v, sem):
    wid = lax.axis_index("s") * 2 + lax.axis_index("c")
    row0 = wid * RPW
    blk0 = wid * CHUNKS

    def one_table(tab, out_hbm, idx_hbm):
        pltpu.sync_copy(idx_hbm.at[pl.ds(blk0, CHUNKS)], idx_v)
        copies = []
        for j in range(CHUNKS):
            copies.append(
                pltpu.async_copy(
                    tab.at[idx_v.at[j]], rows_v.at[pl.ds(j * IDX_W, IDX_W)], sem
                )
            )
        for c in copies:
            c.wait()
        pltpu.sync_copy(rows_v, out_hbm.at[pl.ds(row0, RPW)])

    one_table(gtab, gout, gidx)
    one_table(ktab, kout, kidx)


def _mlp_body(g_ref, k_ref, w1_ref, w2_ref, b1_ref, b2_ref, o_ref):
    dnums = (((1,), (1,)), ((), ()))
    h = lax.dot_general(
        g_ref[...], w1_ref[:, :EMB], dnums, preferred_element_type=jnp.float32
    )
    h = h + lax.dot_general(
        k_ref[...], w1_ref[:, EMB:], dnums, preferred_element_type=jnp.float32
    )
    h = jnp.maximum(h + b1_ref[...], 0.0)
    o_ref[...] = (
        lax.dot_general(h, w2_ref[...], dnums, preferred_element_type=jnp.float32)
        + b2_ref[...]
    )


BLK = 2048


def _mlp(gbuf, kbuf, w1, w2, b1, b2):
    return pl.pallas_call(
        _mlp_body,
        grid=(N // BLK,),
        in_specs=[
            pl.BlockSpec((BLK, EMB), lambda i: (i, 0)),
            pl.BlockSpec((BLK, EMB), lambda i: (i, 0)),
            pl.BlockSpec((H2, 2 * EMB), lambda i: (0, 0)),
            pl.BlockSpec((OUT, H2), lambda i: (0, 0)),
            pl.BlockSpec((1, H2), lambda i: (0, 0)),
            pl.BlockSpec((1, OUT), lambda i: (0, 0)),
        ],
        out_specs=pl.BlockSpec((BLK, OUT), lambda i: (i, 0)),
        out_shape=jax.ShapeDtypeStruct((N, OUT), jnp.float32),
    )(gbuf, kbuf, w1, w2, b1, b2)


def kernel(input_tensor, genre_table, key_table, W1, b1, W2, b2):
    g_idx = input_tensor[:, 0].reshape(N // IDX_W, IDX_W)
    k_idx = input_tensor[:, 1].reshape(N // IDX_W, IDX_W)
    gbuf, kbuf = _sc_gather(genre_table, key_table, g_idx, k_idx)
    return _mlp(gbuf, kbuf, W1, W2, b1.reshape(1, H2), b2.reshape(1, OUT))
